# speedup vs baseline: 1.0304x; 1.0304x over previous
import jax
import jax.numpy as jnp
from jax.experimental import pallas as pl
from jax.experimental.pallas import tpu as pltpu


def _hl_conv(x, ei, ew, W, b):
    src, dst = ei[0], ei[1]
    out = x @ W[0]
    if W.shape[0] > 1:
        Lx = jnp.zeros_like(x).at[dst].add(ew[:, None] * x[src])
        out = out + (x - Lx) @ W[1]
    return out + b


def _bn(x, eps=1e-5):
    m = jnp.mean(x, axis=0, keepdims=True)
    v = jnp.var(x, axis=0, keepdims=True)
    return (x - m) / jnp.sqrt(v + eps)


def _ne_int(x_t, x_s, src, dst, D, Wt, bt, Ws, bs):
    N = x_t.shape[0]
    x_s2t = (jnp.zeros((N, x_s.shape[1]), x_s.dtype).at[src].add(x_s).at[dst].add(x_s)) / D[:, None]
    x_t2s = (x_t[src] + x_t[dst]) * 0.5
    xt = jax.nn.relu(jnp.concatenate([x_t, x_s2t], axis=-1) @ Wt + bt)
    xs = jax.nn.relu(jnp.concatenate([x_s, x_t2s], axis=-1) @ Ws + bs)
    return xt, xs


def _final_mm_kernel(xs_ref, w_ref, b_ref, o_ref):
    o_ref[...] = xs_ref[...] @ w_ref[...] + b_ref[0, 0]


def kernel(x_t, x_s, edge_weight_t, edge_weight_s, Wt_init, bt_init, Ws_init, bs_init,
           Wi0_t, bi0_t, Wi0_s, bi0_s, Wc0_t, bc0_t, Wc0_s, bc0_s,
           Wi1_t, bi1_t, Wi1_s, bi1_s, Wc1_t, bc1_t, Wc1_s, bc1_s,
           W_out, b_out, edge_index_t, edge_index_s, edge_index):
    N = x_t.shape[0]
    src, dst = edge_index[0], edge_index[1]
    D = jnp.zeros((N,), jnp.float32).at[edge_index.reshape(-1)].add(1.0) + 1e-6
    xt = jax.nn.relu(_bn(_hl_conv(x_t, edge_index_t, edge_weight_t, Wt_init, bt_init)))
    xs = jax.nn.relu(_bn(_hl_conv(x_s, edge_index_s, edge_weight_s, Ws_init, bs_init)))
    xt0, xs0 = xt, xs
    xt, xs = _ne_int(xt0, xs0, src, dst, D, Wi0_t, bi0_t, Wi0_s, bi0_s)
    xt = jax.nn.relu(_bn(_hl_conv(xt, edge_index_t, edge_weight_t, Wc0_t, bc0_t)))
    xs = jax.nn.relu(_bn(_hl_conv(xs, edge_index_s, edge_weight_s, Wc0_s, bc0_s)))
    xt0 = jnp.concatenate([xt0, xt], axis=-1)
    xs0 = jnp.concatenate([xs0, xs], axis=-1)
    xt, xs = _ne_int(xt0, xs0, src, dst, D, Wi1_t, bi1_t, Wi1_s, bi1_s)
    xt = jax.nn.relu(_bn(_hl_conv(xt, edge_index_t, edge_weight_t, Wc1_t, bc1_t)))
    xs = jax.nn.relu(_bn(_hl_conv(xs, edge_index_s, edge_weight_s, Wc1_s, bc1_s)))
    x_t2s = (xt[src] + xt[dst]) * 0.5
    xs_cat = jnp.concatenate([xs, x_t2s], axis=-1)

    E = xs_cat.shape[0]
    BLK = 4000
    out = pl.pallas_call(
        _final_mm_kernel,
        grid=(E // BLK,),
        in_specs=[
            pl.BlockSpec((BLK, 128), lambda i: (i, 0)),
            pl.BlockSpec((128, 1), lambda i: (0, 0)),
            pl.BlockSpec((1, 1), lambda i: (0, 0), memory_space=pltpu.SMEM),
        ],
        out_specs=pl.BlockSpec((BLK, 1), lambda i: (i, 0)),
        out_shape=jax.ShapeDtypeStruct((E, 1), jnp.float32),
    )(xs_cat, W_out[0], b_out.reshape(1, 1))
    return out


# speedup vs baseline: 1.8595x; 1.8046x over previous
import functools

import jax
import jax.numpy as jnp
from jax import lax
from jax.experimental import pallas as pl
from jax.experimental.pallas import tpu as pltpu
from jax.experimental.pallas import tpu_sc as plsc

NC, NS, L = 2, 16, 16
NW = NC * NS
BPAD = 320
EPS_BN = 1e-5

_cp = pltpu.CompilerParams(needs_layout_passes=False)
_mesh = plsc.VectorSubcoreMesh(core_axis_name="c", subcore_axis_name="s",
                               num_cores=NC, num_subcores=NS)


def _wid():
    return lax.axis_index("s") * NC + lax.axis_index("c")


def _iota16():
    return lax.iota(jnp.int32, 16)


def _count_fn(nnz, ch, shift, dst_hbm, cnt_hbm, dbuf, hist):
    wid = _wid()
    il = _iota16()
    zeros = jnp.zeros((16,), jnp.int32)
    ones = jnp.full((16,), 1, jnp.int32)

    def z(i, _):
        hist[pl.ds(i * 16, 16)] = zeros
        return 0
    lax.fori_loop(0, (L * BPAD) // 16, z, 0)

    epp = nnz // NW

    def chunk(j, _):
        e0 = wid * epp + j * ch
        pltpu.sync_copy(dst_hbm.at[pl.ds(e0, ch)], dbuf)

        def step(k, _2):
            d = dbuf[pl.ds(k * 16, 16)]
            b = lax.shift_right_logical(d, shift)
            plsc.addupdate_scatter(hist, [il * BPAD + b], ones)
            return 0
        lax.fori_loop(0, ch // 16, step, 0)
        return 0
    lax.fori_loop(0, epp // ch, chunk, 0)
    pltpu.sync_copy(hist, cnt_hbm.at[pl.ds(wid * L * BPAD, L * BPAD)])


def _count(dst, nnz, ch, shift):
    return pl.kernel(
        functools.partial(_count_fn, nnz, ch, shift),
        out_type=jax.ShapeDtypeStruct((NW * L * BPAD,), jnp.int32),
        mesh=_mesh,
        compiler_params=_cp,
        scratch_types=[
            pltpu.VMEM((ch,), jnp.int32),
            pltpu.VMEM((L * BPAD,), jnp.int32),
        ],
    )(dst)


def _prefix_fn(cnt_hbm, off_hbm, rs_hbm, nch_hbm, tot_hbm,
               cbuf, tot, rs, nch, run):
    wid = _wid()
    half = (NW * L * BPAD) // 2
    krows = NW * L // 2

    @pl.when(wid == 0)
    def _():
        zeros = jnp.zeros((16,), jnp.int32)

        def z(i, _):
            tot[pl.ds(i * 16, 16)] = zeros
            run[pl.ds(i * 16, 16)] = zeros
            return 0
        lax.fori_loop(0, BPAD // 16, z, 0)

        for h in range(2):
            pltpu.sync_copy(cnt_hbm.at[pl.ds(h * half, half)], cbuf)

            def acc(k, _):
                for bb in range(BPAD // 16):
                    p = pl.ds(k * BPAD + bb * 16, 16)
                    q = pl.ds(bb * 16, 16)
                    tot[q] = tot[q] + cbuf[p]
                return 0
            lax.fori_loop(0, krows, acc, 0)

        def pre(bb, carry):
            q = pl.ds(bb * 16, 16)
            cap = lax.shift_right_logical(tot[q] + 127, 7)
            csum = plsc.cumsum(cap)
            rs[q] = (carry + csum - cap) * 128
            nch[q] = cap
            return carry + csum[15]
        lax.fori_loop(0, BPAD // 16, pre, jnp.int32(0))

        for h in range(2):
            pltpu.sync_copy(cnt_hbm.at[pl.ds(h * half, half)], cbuf)

            def off(k, _):
                for bb in range(BPAD // 16):
                    p = pl.ds(k * BPAD + bb * 16, 16)
                    q = pl.ds(bb * 16, 16)
                    c = cbuf[p]
                    cbuf[p] = rs[q] + run[q]
                    run[q] = run[q] + c
                return 0
            lax.fori_loop(0, krows, off, 0)
            pltpu.sync_copy(cbuf, off_hbm.at[pl.ds(h * half, half)])

        pltpu.sync_copy(rs, rs_hbm)
        pltpu.sync_copy(nch, nch_hbm)
        pltpu.sync_copy(tot, tot_hbm)


def _prefix(cnt):
    shp = jax.ShapeDtypeStruct
    return pl.kernel(
        _prefix_fn,
        out_type=(shp((NW * L * BPAD,), jnp.int32), shp((BPAD,), jnp.int32),
                  shp((BPAD,), jnp.int32), shp((BPAD,), jnp.int32)),
        mesh=_mesh,
        compiler_params=_cp,
        scratch_types=[
            pltpu.VMEM(((NW * L * BPAD) // 2,), jnp.int32),
            pltpu.VMEM((BPAD,), jnp.int32),
            pltpu.VMEM((BPAD,), jnp.int32),
            pltpu.VMEM((BPAD,), jnp.int32),
            pltpu.VMEM((BPAD,), jnp.int32),
        ],
    )(cnt)


def _place_fn(nnz, ch, shift, rmask, nb, cap,
              gidx_hbm, dst_hbm, ew_hbm, off_hbm, rs_hbm, tot_hbm, nch_hbm,
              recg_hbm, recr_hbm, rece_hbm,
              offs, gbuf, dbuf, ebuf, gvo, rvo, evo, posb, rsv, totv, nchv):
    wid = _wid()
    il = _iota16()
    epp = nnz // NW
    dump = cap - 128

    pltpu.sync_copy(off_hbm.at[pl.ds(wid * L * BPAD, L * BPAD)], offs)

    def flush():
        pltpu.sync_copy(gvo, recg_hbm.at[posb])
        pltpu.sync_copy(rvo, recr_hbm.at[posb])
        pltpu.sync_copy(evo, rece_hbm.at[posb])

    def chunk(j, rp):
        e0 = wid * epp + j * ch
        pltpu.sync_copy(gidx_hbm.at[pl.ds(e0, ch)], gbuf)
        pltpu.sync_copy(dst_hbm.at[pl.ds(e0, ch)], dbuf)
        pltpu.sync_copy(ew_hbm.at[pl.ds(e0, ch)], ebuf)

        def step(k, rp):
            q = pl.ds(k * 16, 16)
            d = dbuf[q]
            b = lax.shift_right_logical(d, shift)
            rel = jnp.bitwise_and(d, rmask)
            fi = il * BPAD + b
            pos = plsc.load_gather(offs, [fi])
            plsc.store_scatter(offs, [fi], pos + 1)
            w = pl.ds(rp, 16)
            posb[w] = pos
            gvo[w] = gbuf[q]
            rvo[w] = rel
            evo[w] = ebuf[q]
            rp = rp + 16

            def do_flush():
                flush()
            pl.when(rp == 128)(do_flush)
            return jnp.where(rp == 128, 0, rp)
        return lax.fori_loop(0, ch // 16, step, rp)

    rp = lax.fori_loop(0, epp // ch, chunk, jnp.int32(0))

    @pl.when(rp > 0)
    def _():
        def pad(k, _):
            posb[pl.ds(rp + k * 16, 16)] = dump + rp + k * 16 + il
            return 0
        lax.fori_loop(0, (128 - rp) // 16, pad, 0)
        flush()

    @pl.when(wid == NW - 1)
    def _():
        pltpu.sync_copy(rs_hbm, rsv.at[pl.ds(0, BPAD)])
        pltpu.sync_copy(tot_hbm, totv.at[pl.ds(0, BPAD)])
        pltpu.sync_copy(nch_hbm, nchv.at[pl.ds(0, BPAD)])
        zi = jnp.zeros((16,), jnp.int32)
        zf = jnp.zeros((16,), jnp.float32)

        def zr(i, _):
            q = pl.ds(i * 16, 16)
            gvo[q] = zi
            rvo[q] = zi
            evo[q] = zf
            return 0
        lax.fori_loop(0, 8, zr, 0)

        def buck(b, _):
            t = totv[pl.ds(b, 16)][0]
            c = nchv[pl.ds(b, 16)][0] * 128
            r0 = rsv[pl.ds(b, 16)][0]
            s = r0 + t
            end = r0 + c

            @pl.when(c > t)
            def _2():
                def grp(g, _3):
                    cand = s + g * 16 + il
                    ok = cand < end
                    posb[pl.ds(g * 16, 16)] = jnp.where(ok, cand,
                                                        dump + g * 16 + il)
                    return 0
                lax.fori_loop(0, 8, grp, 0)
                flush()
            return 0
        lax.fori_loop(0, nb, buck, 0)


def _place(gidx, dst, ew, off, rs, tot, nch, nnz, ch, shift, rmask, nb, cap):
    shp = jax.ShapeDtypeStruct
    return pl.kernel(
        functools.partial(_place_fn, nnz, ch, shift, rmask, nb, cap),
        out_type=(shp((cap,), jnp.int32), shp((cap,), jnp.int32),
                  shp((cap,), jnp.float32)),
        mesh=_mesh,
        compiler_params=_cp,
        scratch_types=[
            pltpu.VMEM((L * BPAD,), jnp.int32),
            pltpu.VMEM((ch,), jnp.int32),
            pltpu.VMEM((ch,), jnp.int32),
            pltpu.VMEM((ch,), jnp.float32),
            pltpu.VMEM((128,), jnp.int32),
            pltpu.VMEM((128,), jnp.int32),
            pltpu.VMEM((128,), jnp.float32),
            pltpu.VMEM((128,), jnp.int32),
            pltpu.VMEM((BPAD + 16,), jnp.int32),
            pltpu.VMEM((BPAD + 16,), jnp.int32),
            pltpu.VMEM((BPAD + 16,), jnp.int32),
        ],
    )(gidx, dst, ew, off, rs, tot, nch)


def _bucketize(gidx, dst, ew, nnz, ch, shift, rmask, nb, cap):
    cnt = _count(dst, nnz, ch, shift)
    off, rs, nch_, tot = _prefix(cnt)
    recg, recr, rece = _place(gidx, dst, ew, off, rs, tot, nch_, nnz, ch,
                              shift, rmask, nb, cap)
    return recg, recr, rece, rs, nch_


def _mp_fn(nb, r_rows, ca, with_ones, *args):
    if with_ones:
        (recg_hbm, recr_hbm, rece_hbm, rs_hbm, nch_hbm, x_hbm,
         out_hbm, oned_hbm,
         slab, slab1, gix, relb, ewb, rows, rsv, nchv, sem) = args
    else:
        (recg_hbm, recr_hbm, rece_hbm, rs_hbm, nch_hbm, x_hbm, out_hbm,
         slab, gix, relb, ewb, rows, rsv, nchv, sem) = args
        slab1 = None
    wid = _wid()
    zf = jnp.zeros((16,), jnp.float32)
    cb_n = ca // 16

    pltpu.sync_copy(rs_hbm, rsv.at[pl.ds(0, BPAD)])
    pltpu.sync_copy(nch_hbm, nchv.at[pl.ds(0, BPAD)])

    for a in range((nb + NW - 1) // NW):
        b = wid + a * NW

        @pl.when(b < nb)
        def _():
            def zrow(i, _):
                for cb in range(8):
                    slab[i, pl.ds(cb * 16, 16)] = zf
                if with_ones:
                    slab1[i, pl.ds(0, 16)] = zf
                return 0
            lax.fori_loop(0, r_rows, zrow, 0)

            rsb = rsv[pl.ds(b, 16)][0]
            ncb = nchv[pl.ds(b, 16)][0]

            def chunk(j, _):
                base = pl.multiple_of(rsb + j * 128, 128)
                pltpu.sync_copy(recg_hbm.at[pl.ds(base, 128)], gix)
                pltpu.sync_copy(recr_hbm.at[pl.ds(base, 128)],
                                relb.at[pl.ds(0, 128)])
                pltpu.sync_copy(rece_hbm.at[pl.ds(base, 128)],
                                ewb.at[pl.ds(0, 128)])
                pltpu.async_copy(x_hbm.at[gix], rows, sem).wait()

                def edge(e, _2):
                    w = ewb[pl.ds(e, 16)][0]
                    rel = relb[pl.ds(e, 16)][0]
                    for cb in range(cb_n):
                        v = rows[e, pl.ds(cb * 16, 16)] * w
                        plsc.addupdate(slab.at[rel, pl.ds(cb * 16, 16)], v)
                    if with_ones:
                        plsc.addupdate(slab1.at[rel, pl.ds(0, 16)],
                                       jnp.full((16,), w))
                    return 0
                lax.fori_loop(0, 128, edge, 0)
                return 0
            lax.fori_loop(0, ncb, chunk, 0)
            o0 = pl.multiple_of(b * r_rows, 8)
            pltpu.sync_copy(slab, out_hbm.at[pl.ds(o0, r_rows)])
            if with_ones:
                pltpu.sync_copy(slab1, oned_hbm.at[pl.ds(o0, r_rows)])


def _mp(recg, recr, rece, rs, nch, x, nb, r_rows, ca, with_ones=False):
    shp = jax.ShapeDtypeStruct
    outs = shp((nb * r_rows, 128), jnp.float32)
    if with_ones:
        outs = (outs, shp((nb * r_rows, 128), jnp.float32))
    scratch = [pltpu.VMEM((r_rows, 128), jnp.float32)]
    if with_ones:
        scratch.append(pltpu.VMEM((r_rows, 128), jnp.float32))
    scratch += [
        pltpu.VMEM((128,), jnp.int32),
        pltpu.VMEM((144,), jnp.int32),
        pltpu.VMEM((144,), jnp.float32),
        pltpu.VMEM((128, 128), jnp.float32),
        pltpu.VMEM((BPAD + 16,), jnp.int32),
        pltpu.VMEM((BPAD + 16,), jnp.int32),
        pltpu.SemaphoreType.DMA,
    ]
    return pl.kernel(
        functools.partial(_mp_fn, nb, r_rows, ca, with_ones),
        out_type=outs,
        mesh=_mesh,
        compiler_params=_cp,
        scratch_types=scratch,
    )(recg, recr, rece, rs, nch, x)


def _t2s_fn(nchunks, ca, x_hbm, src_hbm, dst_hbm, out_hbm,
            sa, sb, ra, rb, sem):
    wid = _wid()
    cb_n = ca // 16

    for a in range((nchunks + NW - 1) // NW):
        ci = wid + a * NW

        @pl.when(ci < nchunks)
        def _():
            e0 = pl.multiple_of(ci * 128, 128)
            pltpu.sync_copy(src_hbm.at[pl.ds(e0, 128)], sa)
            pltpu.sync_copy(dst_hbm.at[pl.ds(e0, 128)], sb)
            cp1 = pltpu.async_copy(x_hbm.at[sa], ra, sem)
            cp2 = pltpu.async_copy(x_hbm.at[sb], rb, sem)
            cp1.wait()
            cp2.wait()

            def add(e, _2):
                for cb in range(cb_n):
                    q = pl.ds(cb * 16, 16)
                    ra[e, q] = ra[e, q] + rb[e, q]
                return 0
            lax.fori_loop(0, 128, add, 0)
            pltpu.sync_copy(ra, out_hbm.at[pl.ds(e0, 128)])


def _t2s(x, src, dst, en, ca):
    return pl.kernel(
        functools.partial(_t2s_fn, en // 128, ca),
        out_type=jax.ShapeDtypeStruct((en, 128), jnp.float32),
        mesh=_mesh,
        compiler_params=_cp,
        scratch_types=[
            pltpu.VMEM((128,), jnp.int32),
            pltpu.VMEM((128,), jnp.int32),
            pltpu.VMEM((128, 128), jnp.float32),
            pltpu.VMEM((128, 128), jnp.float32),
            pltpu.SemaphoreType.DMA,
        ],
    )(x, src, dst)


def _conv1_body(blk, x_ref, lx_ref, ws_ref, w1_ref, b_ref, y_ref, st_ref):
    i = pl.program_id(0)
    y = (jnp.dot(x_ref[...], ws_ref[...], preferred_element_type=jnp.float32,
                   precision=lax.Precision.HIGHEST)
         - jnp.dot(lx_ref[...], w1_ref[...], preferred_element_type=jnp.float32,
                   precision=lax.Precision.HIGHEST)
         + b_ref[...])
    y_ref[...] = y

    m_b = jnp.mean(y, axis=0, keepdims=True)
    m2_b = jnp.sum((y - m_b) * (y - m_b), axis=0, keepdims=True)

    @pl.when(i == 0)
    def _():
        st_ref[...] = jnp.zeros_like(st_ref)
        st_ref[0:1, :] = m_b
        st_ref[1:2, :] = m2_b

    @pl.when(i > 0)
    def _():
        na = (i * blk).astype(jnp.float32)
        nb = float(blk)
        delta = m_b - st_ref[0:1, :]
        st_ref[0:1, :] += delta * (nb / (na + nb))
        st_ref[1:2, :] += m2_b + delta * delta * (na * nb / (na + nb))


def _conv1(x, lx, wsum, w1, b, mreal, blk):
    cin = x.shape[1]
    c = wsum.shape[1]
    nb = mreal // blk
    return pl.pallas_call(
        functools.partial(_conv1_body, blk),
        grid=(nb,),
        in_specs=[
            pl.BlockSpec((blk, cin), lambda i: (i, 0)),
            pl.BlockSpec((blk, cin), lambda i: (i, 0)),
            pl.BlockSpec((cin, c), lambda i: (0, 0)),
            pl.BlockSpec((cin, c), lambda i: (0, 0)),
            pl.BlockSpec((1, c), lambda i: (0, 0)),
        ],
        out_specs=[
            pl.BlockSpec((blk, c), lambda i: (i, 0)),
            pl.BlockSpec((8, c), lambda i: (0, 0)),
        ],
        out_shape=[
            jax.ShapeDtypeStruct((x.shape[0], c), jnp.float32),
            jax.ShapeDtypeStruct((8, c), jnp.float32),
        ],
    )(x, lx, wsum, w1, b.reshape(1, c))


def _conv2_body(mreal, y_ref, st_ref, o_ref):
    m = st_ref[0:1, :]
    v = st_ref[1:2, :] / mreal
    o_ref[...] = jnp.maximum((y_ref[...] - m) / jnp.sqrt(v + EPS_BN), 0.0)


def _conv2(y, st, mreal, blk):
    c = y.shape[1]
    nb = mreal // blk
    return pl.pallas_call(
        functools.partial(_conv2_body, float(mreal)),
        grid=(nb,),
        in_specs=[
            pl.BlockSpec((blk, c), lambda i: (i, 0)),
            pl.BlockSpec((8, c), lambda i: (0, 0)),
        ],
        out_specs=pl.BlockSpec((blk, c), lambda i: (i, 0)),
        out_shape=jax.ShapeDtypeStruct((y.shape[0], c), jnp.float32),
    )(y, st)


def _ne_t_body(x0_ref, s2t_ref, d_ref, wa_ref, wb_ref, b_ref, o_ref):
    inv = 1.0 / (d_ref[...][:, 0:1] + 1e-6)
    o = (jnp.dot(x0_ref[...], wa_ref[...], preferred_element_type=jnp.float32,
                   precision=lax.Precision.HIGHEST)
         + jnp.dot(s2t_ref[...] * inv, wb_ref[...],
                   preferred_element_type=jnp.float32,
                   precision=lax.Precision.HIGHEST)
         + b_ref[...])
    o_ref[...] = jnp.maximum(o, 0.0)


def _ne_t(x0, s2t, d, wa, wb, b, mreal, blk):
    c0, c1 = x0.shape[1], s2t.shape[1]
    c = wa.shape[1]
    return pl.pallas_call(
        _ne_t_body,
        grid=(mreal // blk,),
        in_specs=[
            pl.BlockSpec((blk, c0), lambda i: (i, 0)),
            pl.BlockSpec((blk, c1), lambda i: (i, 0)),
            pl.BlockSpec((blk, 128), lambda i: (i, 0)),
            pl.BlockSpec((c0, c), lambda i: (0, 0)),
            pl.BlockSpec((c1, c), lambda i: (0, 0)),
            pl.BlockSpec((1, c), lambda i: (0, 0)),
        ],
        out_specs=pl.BlockSpec((blk, c), lambda i: (i, 0)),
        out_shape=jax.ShapeDtypeStruct((x0.shape[0], c), jnp.float32),
    )(x0, s2t, d, wa, wb, b.reshape(1, c))


def _ne_s_body(x0_ref, t2s_ref, wa_ref, wb_ref, b_ref, o_ref):
    o = (jnp.dot(x0_ref[...], wa_ref[...], preferred_element_type=jnp.float32,
                   precision=lax.Precision.HIGHEST)
         + jnp.dot(t2s_ref[...], wb_ref[...], preferred_element_type=jnp.float32,
                   precision=lax.Precision.HIGHEST)
         + b_ref[...])
    o_ref[...] = jnp.maximum(o, 0.0)


def _ne_s(x0, t2s, wa, wb, b, mreal, blk, relu=True):
    body = _ne_s_body if relu else _fin_body
    c0, c1 = x0.shape[1], t2s.shape[1]
    c = wa.shape[1]
    return pl.pallas_call(
        body,
        grid=(mreal // blk,),
        in_specs=[
            pl.BlockSpec((blk, c0), lambda i: (i, 0)),
            pl.BlockSpec((blk, c1), lambda i: (i, 0)),
            pl.BlockSpec((c0, c), lambda i: (0, 0)),
            pl.BlockSpec((c1, c), lambda i: (0, 0)),
            pl.BlockSpec((1, c), lambda i: (0, 0)),
        ],
        out_specs=pl.BlockSpec((blk, c), lambda i: (i, 0)),
        out_shape=jax.ShapeDtypeStruct((mreal, c), jnp.float32),
    )(x0, t2s, wa, wb, b.reshape(1, c))


def _fin_body(x0_ref, t2s_ref, wa_ref, wb_ref, b_ref, o_ref):
    o_ref[...] = (jnp.dot(x0_ref[...], wa_ref[...],
                          preferred_element_type=jnp.float32,
                   precision=lax.Precision.HIGHEST)
                  + jnp.dot(t2s_ref[...], wb_ref[...],
                            preferred_element_type=jnp.float32,
                   precision=lax.Precision.HIGHEST)
                  + b_ref[...])


def kernel(x_t, x_s, edge_weight_t, edge_weight_s, Wt_init, bt_init, Ws_init, bs_init,
           Wi0_t, bi0_t, Wi0_s, bi0_s, Wc0_t, bc0_t, Wc0_s, bc0_s,
           Wi1_t, bi1_t, Wi1_s, bi1_s, Wc1_t, bc1_t, Wc1_s, bc1_s,
           W_out, b_out, edge_index_t, edge_index_s, edge_index):
    N = x_t.shape[0]
    E = x_s.shape[0]
    nnz_t = edge_index_t.shape[1]
    nnz_s = edge_index_s.shape[1]

    RT, ST_, BT = 256, 8, (N + 255) // 256
    RS, SS_, BS = 512, 9, (E + 511) // 512
    NP_ = BT * RT
    CAP_T = nnz_t + BT * 128 + 128
    CAP_S = nnz_s + BS * 128 + 128
    CAP_I = 2 * E + BT * 128 + 128

    i32 = jnp.int32
    src_t = edge_index_t[0].astype(i32)
    dst_t = edge_index_t[1].astype(i32)
    src_s = edge_index_s[0].astype(i32)
    dst_s = edge_index_s[1].astype(i32)
    src = edge_index[0].astype(i32)
    dst = edge_index[1].astype(i32)

    inode = jnp.concatenate([src, dst])
    igid = jnp.tile(jnp.arange(E, dtype=i32), 2)
    iew = jnp.ones((2 * E,), jnp.float32)

    rt = _bucketize(src_t, dst_t, edge_weight_t, nnz_t, 2000, ST_, RT - 1,
                    BT, CAP_T)
    rs_ = _bucketize(src_s, dst_s, edge_weight_s, nnz_s, 2000, SS_, RS - 1,
                     BS, CAP_S)
    ri = _bucketize(igid, inode, iew, 2 * E, 2000, ST_, RT - 1, BT, CAP_I)

    xt_pad = jnp.zeros((NP_, 128), jnp.float32).at[:N, :2].set(x_t)
    xs_pad = jnp.zeros((E, 128), jnp.float32).at[:, :1].set(x_s)

    def wp(w):
        return jnp.zeros((128, 128), jnp.float32).at[:w.shape[0],
                                                     :w.shape[1]].set(w)

    def bp(b):
        return jnp.zeros((128,), jnp.float32).at[:b.shape[0]].set(b)

    lx = _mp(*rt, xt_pad, BT, RT, 16)
    y, st = _conv1(xt_pad, lx, wp(Wt_init[0] + Wt_init[1]), wp(Wt_init[1]),
                   bp(bt_init), N, 80)
    xt0 = _conv2(y, st, N, 80)

    lx = _mp(*rs_, xs_pad, BS, RS, 16)
    y, st = _conv1(xs_pad, lx, wp(Ws_init[0] + Ws_init[1]), wp(Ws_init[1]),
                   bp(bs_init), E, 128)
    xs0 = _conv2(y, st, E, 128)

    s2t0, d_arr = _mp(*ri, xs0, BT, RT, 32, with_ones=True)
    t2s0 = _t2s(xt0, src, dst, E, 32)
    xt1 = _ne_t(xt0, s2t0, d_arr, wp(Wi0_t[:32]), wp(Wi0_t[32:]),
                bp(bi0_t), N, 80)
    xs1 = _ne_s(xs0, t2s0, wp(Wi0_s[:32]), wp(0.5 * Wi0_s[32:]),
                bp(bi0_s), E, 128)

    lx = _mp(*rt, xt1, BT, RT, 32)
    y, st = _conv1(xt1, lx, wp(Wc0_t[0] + Wc0_t[1]), wp(Wc0_t[1]),
                   bp(bc0_t), N, 80)
    xt_c0 = _conv2(y, st, N, 80)
    lx = _mp(*rs_, xs1, BS, RS, 32)
    y, st = _conv1(xs1, lx, wp(Wc0_s[0] + Wc0_s[1]), wp(Wc0_s[1]),
                   bp(bc0_s), E, 128)
    xs_c0 = _conv2(y, st, E, 128)

    xt0b = (jnp.zeros((NP_, 128), jnp.float32)
            .at[:, :32].set(xt0[:, :32]).at[:, 32:64].set(xt_c0[:, :32]))
    xs0b = (jnp.zeros((E, 128), jnp.float32)
            .at[:, :32].set(xs0[:, :32]).at[:, 32:64].set(xs_c0[:, :32]))

    s2t1 = _mp(*ri, xs0b, BT, RT, 64)
    t2s1 = _t2s(xt0b, src, dst, E, 64)
    xt1b = _ne_t(xt0b, s2t1, d_arr, wp(Wi1_t[:64]), wp(Wi1_t[64:]),
                 bp(bi1_t), N, 80)
    xs1b = _ne_s(xs0b, t2s1, wp(Wi1_s[:64]), wp(0.5 * Wi1_s[64:]),
                 bp(bi1_s), E, 128)

    lx = _mp(*rt, xt1b, BT, RT, 64)
    y, st = _conv1(xt1b, lx, wp(Wc1_t[0] + Wc1_t[1]), wp(Wc1_t[1]),
                   bp(bc1_t), N, 80)
    xt_f = _conv2(y, st, N, 80)
    lx = _mp(*rs_, xs1b, BS, RS, 64)
    y, st = _conv1(xs1b, lx, wp(Wc1_s[0] + Wc1_s[1]), wp(Wc1_s[1]),
                   bp(bc1_s), E, 128)
    xs_f = _conv2(y, st, E, 128)

    t2s_f = _t2s(xt_f, src, dst, E, 64)
    wa_f = jnp.zeros((128, 1), jnp.float32).at[:64].set(W_out[0][:64])
    wb_f = jnp.zeros((128, 1), jnp.float32).at[:64].set(0.5 * W_out[0][64:])
    out = _ne_s(xs_f, t2s_f, wa_f, wb_f, b_out, E, 128, relu=False)
    return out


# speedup vs baseline: 1.8685x; 1.0049x over previous
import functools

import jax
import jax.numpy as jnp
from jax import lax
from jax.experimental import pallas as pl
from jax.experimental.pallas import tpu as pltpu
from jax.experimental.pallas import tpu_sc as plsc

NC, NS, L = 2, 16, 16
NW = NC * NS
BPAD = 320
EPS_BN = 1e-5

_cp = pltpu.CompilerParams(needs_layout_passes=False)
_mesh = plsc.VectorSubcoreMesh(core_axis_name="c", subcore_axis_name="s",
                               num_cores=NC, num_subcores=NS)


def _wid():
    return lax.axis_index("s") * NC + lax.axis_index("c")


def _iota16():
    return lax.iota(jnp.int32, 16)


def _count_fn(nnz, ch, shift, dst_hbm, cnt_hbm, dbuf, hist):
    wid = _wid()
    il = _iota16()
    zeros = jnp.zeros((16,), jnp.int32)
    ones = jnp.full((16,), 1, jnp.int32)

    def z(i, _):
        hist[pl.ds(i * 16, 16)] = zeros
        return 0
    lax.fori_loop(0, (L * BPAD) // 16, z, 0)

    epp = nnz // NW

    def chunk(j, _):
        e0 = wid * epp + j * ch
        pltpu.sync_copy(dst_hbm.at[pl.ds(e0, ch)], dbuf)

        def step(k, _2):
            d = dbuf[pl.ds(k * 16, 16)]
            b = lax.shift_right_logical(d, shift)
            plsc.addupdate_scatter(hist, [il * BPAD + b], ones)
            return 0
        lax.fori_loop(0, ch // 16, step, 0)
        return 0
    lax.fori_loop(0, epp // ch, chunk, 0)
    pltpu.sync_copy(hist, cnt_hbm.at[pl.ds(wid * L * BPAD, L * BPAD)])


def _count(dst, nnz, ch, shift):
    return pl.kernel(
        functools.partial(_count_fn, nnz, ch, shift),
        out_type=jax.ShapeDtypeStruct((NW * L * BPAD,), jnp.int32),
        mesh=_mesh,
        compiler_params=_cp,
        scratch_types=[
            pltpu.VMEM((ch,), jnp.int32),
            pltpu.VMEM((L * BPAD,), jnp.int32),
        ],
    )(dst)


def _prefix_fn(cnt_hbm, off_hbm, rs_hbm, nch_hbm, tot_hbm,
               cbuf, tot, rs, nch, run):
    wid = _wid()
    half = (NW * L * BPAD) // 2
    krows = NW * L // 2

    @pl.when(wid == 0)
    def _():
        zeros = jnp.zeros((16,), jnp.int32)

        def z(i, _):
            tot[pl.ds(i * 16, 16)] = zeros
            run[pl.ds(i * 16, 16)] = zeros
            return 0
        lax.fori_loop(0, BPAD // 16, z, 0)

        for h in range(2):
            pltpu.sync_copy(cnt_hbm.at[pl.ds(h * half, half)], cbuf)

            def acc(k, _):
                for bb in range(BPAD // 16):
                    p = pl.ds(k * BPAD + bb * 16, 16)
                    q = pl.ds(bb * 16, 16)
                    tot[q] = tot[q] + cbuf[p]
                return 0
            lax.fori_loop(0, krows, acc, 0)

        def pre(bb, carry):
            q = pl.ds(bb * 16, 16)
            cap = lax.shift_right_logical(tot[q] + 127, 7)
            csum = plsc.cumsum(cap)
            rs[q] = (carry + csum - cap) * 128
            nch[q] = cap
            return carry + csum[15]
        lax.fori_loop(0, BPAD // 16, pre, jnp.int32(0))

        for h in range(2):
            pltpu.sync_copy(cnt_hbm.at[pl.ds(h * half, half)], cbuf)

            def off(k, _):
                for bb in range(BPAD // 16):
                    p = pl.ds(k * BPAD + bb * 16, 16)
                    q = pl.ds(bb * 16, 16)
                    c = cbuf[p]
                    cbuf[p] = rs[q] + run[q]
                    run[q] = run[q] + c
                return 0
            lax.fori_loop(0, krows, off, 0)
            pltpu.sync_copy(cbuf, off_hbm.at[pl.ds(h * half, half)])

        pltpu.sync_copy(rs, rs_hbm)
        pltpu.sync_copy(nch, nch_hbm)
        pltpu.sync_copy(tot, tot_hbm)


def _prefix(cnt):
    shp = jax.ShapeDtypeStruct
    return pl.kernel(
        _prefix_fn,
        out_type=(shp((NW * L * BPAD,), jnp.int32), shp((BPAD,), jnp.int32),
                  shp((BPAD,), jnp.int32), shp((BPAD,), jnp.int32)),
        mesh=_mesh,
        compiler_params=_cp,
        scratch_types=[
            pltpu.VMEM(((NW * L * BPAD) // 2,), jnp.int32),
            pltpu.VMEM((BPAD,), jnp.int32),
            pltpu.VMEM((BPAD,), jnp.int32),
            pltpu.VMEM((BPAD,), jnp.int32),
            pltpu.VMEM((BPAD,), jnp.int32),
        ],
    )(cnt)


def _place_fn(nnz, ch, shift, rmask, nb, cap,
              gidx_hbm, dst_hbm, ew_hbm, off_hbm, rs_hbm, tot_hbm, nch_hbm,
              recg_hbm, recr_hbm, rece_hbm,
              offs, gbuf, dbuf, ebuf, gvo, rvo, evo, posb, rsv, totv, nchv):
    wid = _wid()
    il = _iota16()
    epp = nnz // NW
    dump = cap - 128

    pltpu.sync_copy(off_hbm.at[pl.ds(wid * L * BPAD, L * BPAD)], offs)

    def flush():
        pltpu.sync_copy(gvo, recg_hbm.at[posb])
        pltpu.sync_copy(rvo, recr_hbm.at[posb])
        pltpu.sync_copy(evo, rece_hbm.at[posb])

    def chunk(j, rp):
        e0 = wid * epp + j * ch
        pltpu.sync_copy(gidx_hbm.at[pl.ds(e0, ch)], gbuf)
        pltpu.sync_copy(dst_hbm.at[pl.ds(e0, ch)], dbuf)
        pltpu.sync_copy(ew_hbm.at[pl.ds(e0, ch)], ebuf)

        def step(k, rp):
            q = pl.ds(k * 16, 16)
            d = dbuf[q]
            b = lax.shift_right_logical(d, shift)
            rel = jnp.bitwise_and(d, rmask)
            fi = il * BPAD + b
            pos = plsc.load_gather(offs, [fi])
            plsc.store_scatter(offs, [fi], pos + 1)
            w = pl.ds(rp, 16)
            posb[w] = pos
            gvo[w] = gbuf[q]
            rvo[w] = rel
            evo[w] = ebuf[q]
            rp = rp + 16

            def do_flush():
                flush()
            pl.when(rp == 128)(do_flush)
            return jnp.where(rp == 128, 0, rp)
        return lax.fori_loop(0, ch // 16, step, rp)

    rp = lax.fori_loop(0, epp // ch, chunk, jnp.int32(0))

    @pl.when(rp > 0)
    def _():
        def pad(k, _):
            posb[pl.ds(rp + k * 16, 16)] = dump + rp + k * 16 + il
            return 0
        lax.fori_loop(0, (128 - rp) // 16, pad, 0)
        flush()

    pltpu.sync_copy(rs_hbm, rsv.at[pl.ds(0, BPAD)])
    pltpu.sync_copy(tot_hbm, totv.at[pl.ds(0, BPAD)])
    pltpu.sync_copy(nch_hbm, nchv.at[pl.ds(0, BPAD)])
    zi = jnp.zeros((16,), jnp.int32)
    zf = jnp.zeros((16,), jnp.float32)

    def zr(i, _):
        q = pl.ds(i * 16, 16)
        gvo[q] = zi
        rvo[q] = zi
        evo[q] = zf
        return 0
    lax.fori_loop(0, 8, zr, 0)

    for a2 in range((nb + NW - 1) // NW):
        b = wid + a2 * NW

        @pl.when(b < nb)
        def _():
            t = totv[pl.ds(b, 16)][0]
            c = nchv[pl.ds(b, 16)][0] * 128
            r0 = rsv[pl.ds(b, 16)][0]
            s = r0 + t
            end = r0 + c

            @pl.when(c > t)
            def _2():
                def grp(g, _3):
                    cand = s + g * 16 + il
                    ok = cand < end
                    posb[pl.ds(g * 16, 16)] = jnp.where(ok, cand,
                                                        dump + g * 16 + il)
                    return 0
                lax.fori_loop(0, 8, grp, 0)
                flush()


def _place(gidx, dst, ew, off, rs, tot, nch, nnz, ch, shift, rmask, nb, cap):
    shp = jax.ShapeDtypeStruct
    return pl.kernel(
        functools.partial(_place_fn, nnz, ch, shift, rmask, nb, cap),
        out_type=(shp((cap,), jnp.int32), shp((cap,), jnp.int32),
                  shp((cap,), jnp.float32)),
        mesh=_mesh,
        compiler_params=_cp,
        scratch_types=[
            pltpu.VMEM((L * BPAD,), jnp.int32),
            pltpu.VMEM((ch,), jnp.int32),
            pltpu.VMEM((ch,), jnp.int32),
            pltpu.VMEM((ch,), jnp.float32),
            pltpu.VMEM((128,), jnp.int32),
            pltpu.VMEM((128,), jnp.int32),
            pltpu.VMEM((128,), jnp.float32),
            pltpu.VMEM((128,), jnp.int32),
            pltpu.VMEM((BPAD + 16,), jnp.int32),
            pltpu.VMEM((BPAD + 16,), jnp.int32),
            pltpu.VMEM((BPAD + 16,), jnp.int32),
        ],
    )(gidx, dst, ew, off, rs, tot, nch)


def _bucketize(gidx, dst, ew, nnz, ch, shift, rmask, nb, cap):
    cnt = _count(dst, nnz, ch, shift)
    off, rs, nch_, tot = _prefix(cnt)
    recg, recr, rece = _place(gidx, dst, ew, off, rs, tot, nch_, nnz, ch,
                              shift, rmask, nb, cap)
    return recg, recr, rece, rs, nch_


def _mp_fn(nb, r_rows, ca, with_ones, *args):
    if with_ones:
        (recg_hbm, recr_hbm, rece_hbm, rs_hbm, nch_hbm, x_hbm,
         out_hbm, oned_hbm,
         slab, slab1, gix, relb, ewb, rows, rsv, nchv, sem) = args
    else:
        (recg_hbm, recr_hbm, rece_hbm, rs_hbm, nch_hbm, x_hbm, out_hbm,
         slab, gix, relb, ewb, rows, rsv, nchv, sem) = args
        slab1 = None
    wid = _wid()
    zf = jnp.zeros((16,), jnp.float32)
    cb_n = ca // 16

    pltpu.sync_copy(rs_hbm, rsv.at[pl.ds(0, BPAD)])
    pltpu.sync_copy(nch_hbm, nchv.at[pl.ds(0, BPAD)])

    for a in range((nb + NW - 1) // NW):
        b = wid + a * NW

        @pl.when(b < nb)
        def _():
            def zrow(i, _):
                for cb in range(8):
                    slab[i, pl.ds(cb * 16, 16)] = zf
                if with_ones:
                    slab1[i, pl.ds(0, 16)] = zf
                return 0
            lax.fori_loop(0, r_rows, zrow, 0)

            rsb = rsv[pl.ds(b, 16)][0]
            ncb = nchv[pl.ds(b, 16)][0]

            def chunk(j, _):
                base = pl.multiple_of(rsb + j * 128, 128)
                pltpu.sync_copy(recg_hbm.at[pl.ds(base, 128)], gix)
                pltpu.sync_copy(recr_hbm.at[pl.ds(base, 128)],
                                relb.at[pl.ds(0, 128)])
                pltpu.sync_copy(rece_hbm.at[pl.ds(base, 128)],
                                ewb.at[pl.ds(0, 128)])
                pltpu.async_copy(x_hbm.at[gix], rows, sem).wait()

                def edge(e, _2):
                    w = ewb[pl.ds(e, 16)][0]
                    rel = relb[pl.ds(e, 16)][0]
                    for cb in range(cb_n):
                        v = rows[e, pl.ds(cb * 16, 16)] * w
                        plsc.addupdate(slab.at[rel, pl.ds(cb * 16, 16)], v)
                    if with_ones:
                        plsc.addupdate(slab1.at[rel, pl.ds(0, 16)],
                                       jnp.full((16,), w))
                    return 0
                lax.fori_loop(0, 128, edge, 0)
                return 0
            lax.fori_loop(0, ncb, chunk, 0)
            o0 = pl.multiple_of(b * r_rows, 8)
            pltpu.sync_copy(slab, out_hbm.at[pl.ds(o0, r_rows)])
            if with_ones:
                pltpu.sync_copy(slab1, oned_hbm.at[pl.ds(o0, r_rows)])


def _mp(recg, recr, rece, rs, nch, x, nb, r_rows, ca, with_ones=False):
    shp = jax.ShapeDtypeStruct
    outs = shp((nb * r_rows, 128), jnp.float32)
    if with_ones:
        outs = (outs, shp((nb * r_rows, 128), jnp.float32))
    scratch = [pltpu.VMEM((r_rows, 128), jnp.float32)]
    if with_ones:
        scratch.append(pltpu.VMEM((r_rows, 128), jnp.float32))
    scratch += [
        pltpu.VMEM((128,), jnp.int32),
        pltpu.VMEM((144,), jnp.int32),
        pltpu.VMEM((144,), jnp.float32),
        pltpu.VMEM((128, 128), jnp.float32),
        pltpu.VMEM((BPAD + 16,), jnp.int32),
        pltpu.VMEM((BPAD + 16,), jnp.int32),
        pltpu.SemaphoreType.DMA,
    ]
    return pl.kernel(
        functools.partial(_mp_fn, nb, r_rows, ca, with_ones),
        out_type=outs,
        mesh=_mesh,
        compiler_params=_cp,
        scratch_types=scratch,
    )(recg, recr, rece, rs, nch, x)


def _t2s_fn(nchunks, ca, x_hbm, src_hbm, dst_hbm, out_hbm,
            sa, sb, ra, rb, sem):
    wid = _wid()
    cb_n = ca // 16

    for a in range((nchunks + NW - 1) // NW):
        ci = wid + a * NW

        @pl.when(ci < nchunks)
        def _():
            e0 = pl.multiple_of(ci * 128, 128)
            pltpu.sync_copy(src_hbm.at[pl.ds(e0, 128)], sa)
            pltpu.sync_copy(dst_hbm.at[pl.ds(e0, 128)], sb)
            cp1 = pltpu.async_copy(x_hbm.at[sa], ra, sem)
            cp2 = pltpu.async_copy(x_hbm.at[sb], rb, sem)
            cp1.wait()
            cp2.wait()

            def add(e, _2):
                for cb in range(cb_n):
                    q = pl.ds(cb * 16, 16)
                    ra[e, q] = ra[e, q] + rb[e, q]
                return 0
            lax.fori_loop(0, 128, add, 0)
            pltpu.sync_copy(ra, out_hbm.at[pl.ds(e0, 128)])


def _t2s(x, src, dst, en, ca):
    return pl.kernel(
        functools.partial(_t2s_fn, en // 128, ca),
        out_type=jax.ShapeDtypeStruct((en, 128), jnp.float32),
        mesh=_mesh,
        compiler_params=_cp,
        scratch_types=[
            pltpu.VMEM((128,), jnp.int32),
            pltpu.VMEM((128,), jnp.int32),
            pltpu.VMEM((128, 128), jnp.float32),
            pltpu.VMEM((128, 128), jnp.float32),
            pltpu.SemaphoreType.DMA,
        ],
    )(x, src, dst)


def _conv1_body(blk, x_ref, lx_ref, ws_ref, w1_ref, b_ref, y_ref, st_ref):
    i = pl.program_id(0)
    y = (jnp.dot(x_ref[...], ws_ref[...], preferred_element_type=jnp.float32,
                   precision=lax.Precision.HIGHEST)
         - jnp.dot(lx_ref[...], w1_ref[...], preferred_element_type=jnp.float32,
                   precision=lax.Precision.HIGHEST)
         + b_ref[...])
    y_ref[...] = y

    m_b = jnp.mean(y, axis=0, keepdims=True)
    m2_b = jnp.sum((y - m_b) * (y - m_b), axis=0, keepdims=True)

    @pl.when(i == 0)
    def _():
        st_ref[...] = jnp.zeros_like(st_ref)
        st_ref[0:1, :] = m_b
        st_ref[1:2, :] = m2_b

    @pl.when(i > 0)
    def _():
        na = (i * blk).astype(jnp.float32)
        nb = float(blk)
        delta = m_b - st_ref[0:1, :]
        st_ref[0:1, :] += delta * (nb / (na + nb))
        st_ref[1:2, :] += m2_b + delta * delta * (na * nb / (na + nb))


def _conv1(x, lx, wsum, w1, b, mreal, blk):
    cin = x.shape[1]
    c = wsum.shape[1]
    nb = mreal // blk
    return pl.pallas_call(
        functools.partial(_conv1_body, blk),
        grid=(nb,),
        in_specs=[
            pl.BlockSpec((blk, cin), lambda i: (i, 0)),
            pl.BlockSpec((blk, cin), lambda i: (i, 0)),
            pl.BlockSpec((cin, c), lambda i: (0, 0)),
            pl.BlockSpec((cin, c), lambda i: (0, 0)),
            pl.BlockSpec((1, c), lambda i: (0, 0)),
        ],
        out_specs=[
            pl.BlockSpec((blk, c), lambda i: (i, 0)),
            pl.BlockSpec((8, c), lambda i: (0, 0)),
        ],
        out_shape=[
            jax.ShapeDtypeStruct((x.shape[0], c), jnp.float32),
            jax.ShapeDtypeStruct((8, c), jnp.float32),
        ],
    )(x, lx, wsum, w1, b.reshape(1, c))


def _conv2_body(mreal, y_ref, st_ref, o_ref):
    m = st_ref[0:1, :]
    v = st_ref[1:2, :] / mreal
    o_ref[...] = jnp.maximum((y_ref[...] - m) / jnp.sqrt(v + EPS_BN), 0.0)


def _conv2(y, st, mreal, blk):
    c = y.shape[1]
    nb = mreal // blk
    return pl.pallas_call(
        functools.partial(_conv2_body, float(mreal)),
        grid=(nb,),
        in_specs=[
            pl.BlockSpec((blk, c), lambda i: (i, 0)),
            pl.BlockSpec((8, c), lambda i: (0, 0)),
        ],
        out_specs=pl.BlockSpec((blk, c), lambda i: (i, 0)),
        out_shape=jax.ShapeDtypeStruct((y.shape[0], c), jnp.float32),
    )(y, st)


def _ne_t_body(x0_ref, s2t_ref, d_ref, wa_ref, wb_ref, b_ref, o_ref):
    inv = 1.0 / (d_ref[...][:, 0:1] + 1e-6)
    o = (jnp.dot(x0_ref[...], wa_ref[...], preferred_element_type=jnp.float32,
                   precision=lax.Precision.HIGHEST)
         + jnp.dot(s2t_ref[...] * inv, wb_ref[...],
                   preferred_element_type=jnp.float32,
                   precision=lax.Precision.HIGHEST)
         + b_ref[...])
    o_ref[...] = jnp.maximum(o, 0.0)


def _ne_t(x0, s2t, d, wa, wb, b, mreal, blk):
    c0, c1 = x0.shape[1], s2t.shape[1]
    c = wa.shape[1]
    return pl.pallas_call(
        _ne_t_body,
        grid=(mreal // blk,),
        in_specs=[
            pl.BlockSpec((blk, c0), lambda i: (i, 0)),
            pl.BlockSpec((blk, c1), lambda i: (i, 0)),
            pl.BlockSpec((blk, 128), lambda i: (i, 0)),
            pl.BlockSpec((c0, c), lambda i: (0, 0)),
            pl.BlockSpec((c1, c), lambda i: (0, 0)),
            pl.BlockSpec((1, c), lambda i: (0, 0)),
        ],
        out_specs=pl.BlockSpec((blk, c), lambda i: (i, 0)),
        out_shape=jax.ShapeDtypeStruct((x0.shape[0], c), jnp.float32),
    )(x0, s2t, d, wa, wb, b.reshape(1, c))


def _ne_s_body(x0_ref, t2s_ref, wa_ref, wb_ref, b_ref, o_ref):
    o = (jnp.dot(x0_ref[...], wa_ref[...], preferred_element_type=jnp.float32,
                   precision=lax.Precision.HIGHEST)
         + jnp.dot(t2s_ref[...], wb_ref[...], preferred_element_type=jnp.float32,
                   precision=lax.Precision.HIGHEST)
         + b_ref[...])
    o_ref[...] = jnp.maximum(o, 0.0)


def _ne_s(x0, t2s, wa, wb, b, mreal, blk, relu=True):
    body = _ne_s_body if relu else _fin_body
    c0, c1 = x0.shape[1], t2s.shape[1]
    c = wa.shape[1]
    return pl.pallas_call(
        body,
        grid=(mreal // blk,),
        in_specs=[
            pl.BlockSpec((blk, c0), lambda i: (i, 0)),
            pl.BlockSpec((blk, c1), lambda i: (i, 0)),
            pl.BlockSpec((c0, c), lambda i: (0, 0)),
            pl.BlockSpec((c1, c), lambda i: (0, 0)),
            pl.BlockSpec((1, c), lambda i: (0, 0)),
        ],
        out_specs=pl.BlockSpec((blk, c), lambda i: (i, 0)),
        out_shape=jax.ShapeDtypeStruct((mreal, c), jnp.float32),
    )(x0, t2s, wa, wb, b.reshape(1, c))


def _fin_body(x0_ref, t2s_ref, wa_ref, wb_ref, b_ref, o_ref):
    o_ref[...] = (jnp.dot(x0_ref[...], wa_ref[...],
                          preferred_element_type=jnp.float32,
                   precision=lax.Precision.HIGHEST)
                  + jnp.dot(t2s_ref[...], wb_ref[...],
                            preferred_element_type=jnp.float32,
                   precision=lax.Precision.HIGHEST)
                  + b_ref[...])


def kernel(x_t, x_s, edge_weight_t, edge_weight_s, Wt_init, bt_init, Ws_init, bs_init,
           Wi0_t, bi0_t, Wi0_s, bi0_s, Wc0_t, bc0_t, Wc0_s, bc0_s,
           Wi1_t, bi1_t, Wi1_s, bi1_s, Wc1_t, bc1_t, Wc1_s, bc1_s,
           W_out, b_out, edge_index_t, edge_index_s, edge_index):
    N = x_t.shape[0]
    E = x_s.shape[0]
    nnz_t = edge_index_t.shape[1]
    nnz_s = edge_index_s.shape[1]

    RT, ST_, BT = 256, 8, (N + 255) // 256
    RS, SS_, BS = 512, 9, (E + 511) // 512
    NP_ = BT * RT
    CAP_T = nnz_t + BT * 128 + 128
    CAP_S = nnz_s + BS * 128 + 128
    CAP_I = 2 * E + BT * 128 + 128

    i32 = jnp.int32
    src_t = edge_index_t[0].astype(i32)
    dst_t = edge_index_t[1].astype(i32)
    src_s = edge_index_s[0].astype(i32)
    dst_s = edge_index_s[1].astype(i32)
    src = edge_index[0].astype(i32)
    dst = edge_index[1].astype(i32)

    inode = jnp.concatenate([src, dst])
    igid = jnp.tile(jnp.arange(E, dtype=i32), 2)
    iew = jnp.ones((2 * E,), jnp.float32)

    rt = _bucketize(src_t, dst_t, edge_weight_t, nnz_t, 2000, ST_, RT - 1,
                    BT, CAP_T)
    rs_ = _bucketize(src_s, dst_s, edge_weight_s, nnz_s, 2000, SS_, RS - 1,
                     BS, CAP_S)
    ri = _bucketize(igid, inode, iew, 2 * E, 2000, ST_, RT - 1, BT, CAP_I)

    xt_pad = jnp.zeros((NP_, 128), jnp.float32).at[:N, :2].set(x_t)
    xs_pad = jnp.zeros((E, 128), jnp.float32).at[:, :1].set(x_s)

    def wp(w):
        return jnp.zeros((128, 128), jnp.float32).at[:w.shape[0],
                                                     :w.shape[1]].set(w)

    def bp(b):
        return jnp.zeros((128,), jnp.float32).at[:b.shape[0]].set(b)

    lx = _mp(*rt, xt_pad, BT, RT, 16)
    y, st = _conv1(xt_pad, lx, wp(Wt_init[0] + Wt_init[1]), wp(Wt_init[1]),
                   bp(bt_init), N, 80)
    xt0 = _conv2(y, st, N, 80)

    lx = _mp(*rs_, xs_pad, BS, RS, 16)
    y, st = _conv1(xs_pad, lx, wp(Ws_init[0] + Ws_init[1]), wp(Ws_init[1]),
                   bp(bs_init), E, 128)
    xs0 = _conv2(y, st, E, 128)

    s2t0, d_arr = _mp(*ri, xs0, BT, RT, 32, with_ones=True)
    t2s0 = _t2s(xt0, src, dst, E, 32)
    xt1 = _ne_t(xt0, s2t0, d_arr, wp(Wi0_t[:32]), wp(Wi0_t[32:]),
                bp(bi0_t), N, 80)
    xs1 = _ne_s(xs0, t2s0, wp(Wi0_s[:32]), wp(0.5 * Wi0_s[32:]),
                bp(bi0_s), E, 128)

    lx = _mp(*rt, xt1, BT, RT, 32)
    y, st = _conv1(xt1, lx, wp(Wc0_t[0] + Wc0_t[1]), wp(Wc0_t[1]),
                   bp(bc0_t), N, 80)
    xt_c0 = _conv2(y, st, N, 80)
    lx = _mp(*rs_, xs1, BS, RS, 32)
    y, st = _conv1(xs1, lx, wp(Wc0_s[0] + Wc0_s[1]), wp(Wc0_s[1]),
                   bp(bc0_s), E, 128)
    xs_c0 = _conv2(y, st, E, 128)

    xt0b = (jnp.zeros((NP_, 128), jnp.float32)
            .at[:, :32].set(xt0[:, :32]).at[:, 32:64].set(xt_c0[:, :32]))
    xs0b = (jnp.zeros((E, 128), jnp.float32)
            .at[:, :32].set(xs0[:, :32]).at[:, 32:64].set(xs_c0[:, :32]))

    s2t1 = _mp(*ri, xs0b, BT, RT, 64)
    t2s1 = _t2s(xt0b, src, dst, E, 64)
    xt1b = _ne_t(xt0b, s2t1, d_arr, wp(Wi1_t[:64]), wp(Wi1_t[64:]),
                 bp(bi1_t), N, 80)
    xs1b = _ne_s(xs0b, t2s1, wp(Wi1_s[:64]), wp(0.5 * Wi1_s[64:]),
                 bp(bi1_s), E, 128)

    lx = _mp(*rt, xt1b, BT, RT, 64)
    y, st = _conv1(xt1b, lx, wp(Wc1_t[0] + Wc1_t[1]), wp(Wc1_t[1]),
                   bp(bc1_t), N, 80)
    xt_f = _conv2(y, st, N, 80)
    lx = _mp(*rs_, xs1b, BS, RS, 64)
    y, st = _conv1(xs1b, lx, wp(Wc1_s[0] + Wc1_s[1]), wp(Wc1_s[1]),
                   bp(bc1_s), E, 128)
    xs_f = _conv2(y, st, E, 128)

    t2s_f = _t2s(xt_f, src, dst, E, 64)
    wa_f = jnp.zeros((128, 1), jnp.float32).at[:64].set(W_out[0][:64])
    wb_f = jnp.zeros((128, 1), jnp.float32).at[:64].set(0.5 * W_out[0][64:])
    out = _ne_s(xs_f, t2s_f, wa_f, wb_f, b_out, E, 128, relu=False)
    return out
